# baseline (speedup 1.0000x reference)
import jax
import jax.numpy as jnp
from jax import lax
from jax.experimental import pallas as pl
from jax.experimental.pallas import tpu as pltpu

_EPS = 1e-5
_BF16 = jnp.bfloat16
_F32 = jnp.float32


def _gn1(x, gamma, beta):
    mu = jnp.mean(x, axis=-1, keepdims=True)
    var = jnp.mean((x - mu) ** 2, axis=-1, keepdims=True)
    return (x - mu) * lax.rsqrt(var + _EPS) * gamma + beta


def _round_up(x, m):
    return ((x + m - 1) // m) * m


def _edge_kernel(dist_ref, agt_hi_ref, ctx_wi_ref,
                 vec_ctx_ref, w_dq_ref, w_cat_ref, w_c2_ref, vec_agt_ref,
                 out_ref):
    n_ctx = vec_ctx_ref.shape[1]
    vc = vec_ctx_ref[...]
    wd1_0, wd1_1 = vc[0:1, :], vc[1:2, :]
    b_d1, g_d2, be_d2 = vc[2:3, :], vc[3:4, :], vc[4:5, :]
    g_q, be_q = vc[5:6, :], vc[6:7, :]
    va = vec_agt_ref[...]
    g_c1, be_c1 = va[0:1, :], va[1:2, :]

    dist = dist_ref[...]
    d = dist[:, 0:1] * wd1_0 + dist[:, 1:2] * wd1_1 + b_d1
    d = jnp.maximum(d, 0.0)

    lhs = jnp.concatenate([d.astype(_BF16), agt_hi_ref[...]], axis=-1)
    dq = jnp.dot(lhs, w_dq_ref[...], preferred_element_type=_F32)
    d2 = jnp.maximum(_gn1(dq[:, :n_ctx], g_d2, be_d2), 0.0)
    q = jnp.maximum(_gn1(dq[:, n_ctx:], g_q, be_q), 0.0)

    cat = jnp.concatenate(
        [d2.astype(_BF16), q.astype(_BF16), ctx_wi_ref[...]], axis=-1)
    c = jnp.dot(cat, w_cat_ref[...], preferred_element_type=_F32)
    c = jnp.maximum(_gn1(c, g_c1, be_c1), 0.0)

    out_ref[...] = jnp.dot(c.astype(_BF16), w_c2_ref[...],
                           preferred_element_type=_F32)


def _agt_kernel(agts_ref, added_ref, wagt_ref, wl_ref, vec_ref, out_ref):
    v = vec_ref[...]
    g_n, be_n, g_l, be_l = v[0:1, :], v[1:2, :], v[2:3, :], v[3:4, :]

    res = agts_ref[...]
    x = jnp.dot(res.astype(_BF16), wagt_ref[...],
                preferred_element_type=_F32) + added_ref[...]
    x = jnp.maximum(_gn1(x, g_n, be_n), 0.0)
    x = jnp.dot(x.astype(_BF16), wl_ref[...], preferred_element_type=_F32)
    x = _gn1(x, g_l, be_l)
    out_ref[...] = jnp.maximum(x + res, 0.0)


def _full_spec(shape):
    return pl.BlockSpec(shape, lambda i: (0,) * len(shape))


def _edge_forward(dist, agt_hi, ctx_wi, p, tile_m=1024):
    M = dist.shape[0]
    n_ctx = p["w_d2"].shape[0]
    n_agt = p["w_agt"].shape[0]
    assert M % tile_m == 0

    def row(c):
        return pl.BlockSpec((tile_m, c), lambda i: (i, 0))

    vec_ctx = jnp.concatenate(
        [p["w_d1"].T, p["b_d1"], p["g_d2"], p["be_d2"], p["g_q"], p["be_q"]],
        axis=0)
    vec_agt = jnp.concatenate([p["g_c1"], p["be_c1"]], axis=0)
    zc = jnp.zeros((n_ctx, n_ctx), _BF16)
    w_dq = jnp.concatenate([
        jnp.concatenate([p["w_d2"].T.astype(_BF16), zc], axis=1),
        jnp.concatenate([zc, p["w_q"].T.astype(_BF16)], axis=1)], axis=0)
    w_cat = jnp.concatenate(
        [p["w_c1d"].T, p["w_c1q"].T, p["w_c1c"].T], axis=0).astype(_BF16)
    weights = [vec_ctx, w_dq, w_cat, p["w_c2"].T.astype(_BF16), vec_agt]

    out = pl.pallas_call(
        _edge_kernel,
        out_shape=jax.ShapeDtypeStruct((M, n_agt), _F32),
        grid=(M // tile_m,),
        in_specs=[row(2), row(n_agt), row(n_ctx)]
                 + [_full_spec(w.shape) for w in weights],
        out_specs=row(n_agt),
        compiler_params=pltpu.CompilerParams(dimension_semantics=("parallel",)),
    )(dist, agt_hi, ctx_wi, *weights)
    return out


def _agt_forward(agts, added, p, tile_n=1024):
    N, n_agt = agts.shape
    assert N % tile_n == 0

    row = pl.BlockSpec((tile_n, n_agt), lambda i: (i, 0))
    vec = jnp.concatenate([p["g_n"], p["be_n"], p["g_l"], p["be_l"]], axis=0)
    weights = [p["w_agt"].T.astype(_BF16), p["w_l"].T.astype(_BF16), vec]

    out = pl.pallas_call(
        _agt_kernel,
        out_shape=jax.ShapeDtypeStruct((N, n_agt), _F32),
        grid=(N // tile_n,),
        in_specs=[row, row] + [_full_spec(w.shape) for w in weights],
        out_specs=row,
        compiler_params=pltpu.CompilerParams(dimension_semantics=("parallel",)),
    )(agts, added, *weights)
    return out


@jax.jit
def _att_forward(agts, agt_ctrs_cat, ctx, ctx_ctrs_cat, hi, wi, p):
    dist = agt_ctrs_cat[hi] - ctx_ctrs_cat[wi]
    agt_hi = agts.astype(_BF16)[hi]
    ctx_wi = ctx.astype(_BF16)[wi]
    ctx_out = _edge_forward(dist, agt_hi, ctx_wi, p)
    added = jnp.zeros_like(agts).at[hi].add(ctx_out)
    return _agt_forward(agts, added, p)


def kernel(agts, ctx, agt_ctrs_cat, ctx_ctrs_cat, hi, wi,
           w_d1, b_d1, w_d2, g_d2, be_d2, w_q, g_q, be_q,
           w_c1d, w_c1q, w_c1c, g_c1, be_c1, w_c2, w_agt,
           g_n, be_n, w_l, g_l, be_l):
    p = {
        "w_d1": w_d1, "b_d1": b_d1, "w_d2": w_d2, "g_d2": g_d2, "be_d2": be_d2,
        "w_q": w_q, "g_q": g_q, "be_q": be_q,
        "w_c1d": w_c1d, "w_c1q": w_c1q, "w_c1c": w_c1c,
        "g_c1": g_c1, "be_c1": be_c1, "w_c2": w_c2,
        "w_agt": w_agt, "g_n": g_n, "be_n": be_n,
        "w_l": w_l, "g_l": g_l, "be_l": be_l,
    }
    return _att_forward(agts, agt_ctrs_cat, ctx, ctx_ctrs_cat, hi, wi, p)


# speedup vs baseline: 1.1238x; 1.1238x over previous
import jax
import jax.numpy as jnp
from jax import lax
from jax.experimental import pallas as pl
from jax.experimental.pallas import tpu as pltpu

_EPS = 1e-5
_BF16 = jnp.bfloat16
_F32 = jnp.float32

_TILE = 1024
_WIN = 512


def _gn1(x, gamma, beta):
    mu = jnp.mean(x, axis=-1, keepdims=True)
    var = jnp.mean((x - mu) ** 2, axis=-1, keepdims=True)
    return (x - mu) * lax.rsqrt(var + _EPS) * gamma + beta


def _edge_kernel(ws_ref, flag_ref, dist_ref, agt_hi_ref, ctx_wi_ref,
                 hiv_ref, his_ref,
                 vec_ctx_ref, w_dq_ref, w_cat_ref, w_c2_ref, vec_agt_ref,
                 acc_ref, feat_ref):
    n_ctx = vec_ctx_ref.shape[1]
    nblk = pl.num_programs(1)
    c = pl.program_id(0)
    j = pl.program_id(1)
    b = c * nblk + j

    @pl.when(j == 0)
    def _init():
        acc_ref[...] = jnp.zeros_like(acc_ref)

    vc = vec_ctx_ref[...]
    wd1_0, wd1_1 = vc[0:1, :], vc[1:2, :]
    b_d1, g_d2, be_d2 = vc[2:3, :], vc[3:4, :], vc[4:5, :]
    g_q, be_q = vc[5:6, :], vc[6:7, :]
    va = vec_agt_ref[...]
    g_c1, be_c1 = va[0:1, :], va[1:2, :]

    dist = dist_ref[...]
    d = dist[:, 0:1] * wd1_0 + dist[:, 1:2] * wd1_1 + b_d1
    d = jnp.maximum(d, 0.0)

    lhs = jnp.concatenate([d.astype(_BF16), agt_hi_ref[...]], axis=-1)
    dq = jnp.dot(lhs, w_dq_ref[...], preferred_element_type=_F32)
    d2 = jnp.maximum(_gn1(dq[:, :n_ctx], g_d2, be_d2), 0.0)
    q = jnp.maximum(_gn1(dq[:, n_ctx:], g_q, be_q), 0.0)

    cat = jnp.concatenate(
        [d2.astype(_BF16), q.astype(_BF16), ctx_wi_ref[...]], axis=-1)
    cmid = jnp.dot(cat, w_cat_ref[...], preferred_element_type=_F32)
    cmid = jnp.maximum(_gn1(cmid, g_c1, be_c1), 0.0)
    feat = jnp.dot(cmid.astype(_BF16), w_c2_ref[...],
                   preferred_element_type=_F32)

    ws = pl.multiple_of(ws_ref[b], 8)
    flag = flag_ref[b]
    hiv = hiv_ref[0]

    @pl.when(flag == 0)
    def _onehot_scatter():
        iota = lax.broadcasted_iota(jnp.int32, (_WIN, _TILE), 0)
        oh = (iota + ws == hiv).astype(_BF16)
        partial = jnp.dot(oh, feat.astype(_BF16), preferred_element_type=_F32)
        cur = acc_ref[0, pl.ds(ws, _WIN), :]
        acc_ref[0, pl.ds(ws, _WIN), :] = cur + partial

    @pl.when(flag != 0)
    def _row_scatter():
        feat_ref[...] = feat

        def body(qi, _):
            chunk = feat_ref[pl.ds(qi * 8, 8), :]
            for r in range(8):
                idx = his_ref[0, 0, qi * 8 + r]
                base = pl.multiple_of((idx >> 3) << 3, 8)
                sub = idx & 7
                mask = (lax.broadcasted_iota(jnp.int32, (8, 1), 0)
                        == sub).astype(_F32)
                cur = acc_ref[0, pl.ds(base, 8), :]
                acc_ref[0, pl.ds(base, 8), :] = cur + mask * chunk[r:r + 1, :]
            return 0

        lax.fori_loop(0, _TILE // 8, body, 0)


def _agt_kernel(acc_ref, agts_ref, wagt_ref, wl_ref, vec_ref, out_ref):
    v = vec_ref[...]
    g_n, be_n, g_l, be_l = v[0:1, :], v[1:2, :], v[2:3, :], v[3:4, :]

    res = agts_ref[...]
    added = acc_ref[0] + acc_ref[1]
    x = jnp.dot(res.astype(_BF16), wagt_ref[...],
                preferred_element_type=_F32) + added
    x = jnp.maximum(_gn1(x, g_n, be_n), 0.0)
    x = jnp.dot(x.astype(_BF16), wl_ref[...], preferred_element_type=_F32)
    x = _gn1(x, g_l, be_l)
    out_ref[...] = jnp.maximum(x + res, 0.0)


def _full_spec(shape):
    return pl.BlockSpec(shape, lambda c, j, ws, fl: (0,) * len(shape))


@jax.jit
def _att_forward(agts, agt_ctrs_cat, ctx, ctx_ctrs_cat, hi, wi, p):
    E = hi.shape[0]
    N, n_agt = agts.shape
    n_ctx = p["w_d2"].shape[0]
    nb = E // _TILE
    nblk = nb // 2

    hi_s, perm = lax.sort_key_val(hi, jnp.arange(E, dtype=jnp.int32))
    wi_s = wi[perm]
    dist = agt_ctrs_cat[hi_s] - ctx_ctrs_cat[wi_s]
    agt_hi = agts.astype(_BF16)[hi_s]
    ctx_wi = ctx.astype(_BF16)[wi_s]

    starts = hi_s[::_TILE]
    ws = jnp.minimum((starts >> 3) << 3, N - _WIN)
    last = hi_s[_TILE - 1::_TILE]
    flag = (last - ws >= _WIN).astype(jnp.int32)

    hiv = hi_s.reshape(nb, 1, _TILE)
    his = hi_s.reshape(nb, 1, _TILE)

    vec_ctx = jnp.concatenate(
        [p["w_d1"].T, p["b_d1"], p["g_d2"], p["be_d2"], p["g_q"], p["be_q"]],
        axis=0)
    vec_agt = jnp.concatenate([p["g_c1"], p["be_c1"]], axis=0)
    zc = jnp.zeros((n_ctx, n_ctx), _BF16)
    w_dq = jnp.concatenate([
        jnp.concatenate([p["w_d2"].T.astype(_BF16), zc], axis=1),
        jnp.concatenate([zc, p["w_q"].T.astype(_BF16)], axis=1)], axis=0)
    w_cat = jnp.concatenate(
        [p["w_c1d"].T, p["w_c1q"].T, p["w_c1c"].T], axis=0).astype(_BF16)
    weights = [vec_ctx, w_dq, w_cat, p["w_c2"].T.astype(_BF16), vec_agt]

    def row(ncol):
        return pl.BlockSpec((_TILE, ncol), lambda c, j, ws, fl: (c * nblk + j, 0))

    grid_spec = pltpu.PrefetchScalarGridSpec(
        num_scalar_prefetch=2,
        grid=(2, nblk),
        in_specs=[
            row(2), row(n_agt), row(n_ctx),
            pl.BlockSpec((1, 1, _TILE), lambda c, j, ws, fl: (c * nblk + j, 0, 0)),
            pl.BlockSpec((1, 1, _TILE), lambda c, j, ws, fl: (c * nblk + j, 0, 0),
                         memory_space=pltpu.SMEM),
        ] + [_full_spec(w.shape) for w in weights],
        out_specs=pl.BlockSpec((1, N, n_agt), lambda c, j, ws, fl: (c, 0, 0)),
        scratch_shapes=[pltpu.VMEM((_TILE, n_agt), _F32)],
    )
    acc = pl.pallas_call(
        _edge_kernel,
        grid_spec=grid_spec,
        out_shape=jax.ShapeDtypeStruct((2, N, n_agt), _F32),
        compiler_params=pltpu.CompilerParams(
            dimension_semantics=("parallel", "arbitrary")),
    )(ws, flag, dist, agt_hi, ctx_wi, hiv, his, *weights)

    tile_n = 1024
    vec = jnp.concatenate([p["g_n"], p["be_n"], p["g_l"], p["be_l"]], axis=0)
    out = pl.pallas_call(
        _agt_kernel,
        out_shape=jax.ShapeDtypeStruct((N, n_agt), _F32),
        grid=(N // tile_n,),
        in_specs=[
            pl.BlockSpec((2, tile_n, n_agt), lambda i: (0, i, 0)),
            pl.BlockSpec((tile_n, n_agt), lambda i: (i, 0)),
            pl.BlockSpec((n_agt, n_agt), lambda i: (0, 0)),
            pl.BlockSpec((n_agt, n_agt), lambda i: (0, 0)),
            pl.BlockSpec((4, n_agt), lambda i: (0, 0)),
        ],
        out_specs=pl.BlockSpec((tile_n, n_agt), lambda i: (i, 0)),
        compiler_params=pltpu.CompilerParams(
            dimension_semantics=("parallel",)),
    )(acc, agts, p["w_agt"].T.astype(_BF16), p["w_l"].T.astype(_BF16), vec)
    return out


def kernel(agts, ctx, agt_ctrs_cat, ctx_ctrs_cat, hi, wi,
           w_d1, b_d1, w_d2, g_d2, be_d2, w_q, g_q, be_q,
           w_c1d, w_c1q, w_c1c, g_c1, be_c1, w_c2, w_agt,
           g_n, be_n, w_l, g_l, be_l):
    p = {
        "w_d1": w_d1, "b_d1": b_d1, "w_d2": w_d2, "g_d2": g_d2, "be_d2": be_d2,
        "w_q": w_q, "g_q": g_q, "be_q": be_q,
        "w_c1d": w_c1d, "w_c1q": w_c1q, "w_c1c": w_c1c,
        "g_c1": g_c1, "be_c1": be_c1, "w_c2": w_c2,
        "w_agt": w_agt, "g_n": g_n, "be_n": be_n,
        "w_l": w_l, "g_l": g_l, "be_l": be_l,
    }
    return _att_forward(agts, agt_ctrs_cat, ctx, ctx_ctrs_cat, hi, wi, p)


# speedup vs baseline: 1.7609x; 1.5670x over previous
import jax
import jax.numpy as jnp
from jax import lax
from jax.experimental import pallas as pl
from jax.experimental.pallas import tpu as pltpu

_EPS = 1e-5
_BF16 = jnp.bfloat16
_F32 = jnp.float32

_TILE = 1024
_WIN = 512


def _gn1(x, gamma, beta):
    mu = jnp.mean(x, axis=-1, keepdims=True)
    var = jnp.mean((x - mu) ** 2, axis=-1, keepdims=True)
    return (x - mu) * lax.rsqrt(var + _EPS) * gamma + beta


def _edge_kernel(ws_ref, flag_ref, ag_ref, cg_ref,
                 hiv_ref, his_ref,
                 vec_ctx_ref, w_dq_ref, w_cat_ref, w_c2_ref, vec_agt_ref,
                 acc_ref, feat_ref):
    n_ctx = vec_ctx_ref.shape[1]
    nblk = pl.num_programs(1)
    c = pl.program_id(0)
    j = pl.program_id(1)
    b = c * nblk + j

    @pl.when(j == 0)
    def _init():
        acc_ref[...] = jnp.zeros_like(acc_ref)

    vc = vec_ctx_ref[...]
    wd1_0, wd1_1 = vc[0:1, :], vc[1:2, :]
    b_d1, g_d2, be_d2 = vc[2:3, :], vc[3:4, :], vc[4:5, :]
    g_q, be_q = vc[5:6, :], vc[6:7, :]
    va = vec_agt_ref[...]
    g_c1, be_c1 = va[0:1, :], va[1:2, :]

    na = ag_ref.shape[1] - 2
    nc = cg_ref.shape[1] - 2
    ag = ag_ref[...]
    cg = cg_ref[...]
    dist = (ag[:, na:].astype(_F32) - cg[:, nc:].astype(_F32))
    d = dist[:, 0:1] * wd1_0 + dist[:, 1:2] * wd1_1 + b_d1
    d = jnp.maximum(d, 0.0)

    lhs = jnp.concatenate([d.astype(_BF16), ag[:, :na]], axis=-1)
    dq = jnp.dot(lhs, w_dq_ref[...], preferred_element_type=_F32)
    d2 = jnp.maximum(_gn1(dq[:, :n_ctx], g_d2, be_d2), 0.0)
    q = jnp.maximum(_gn1(dq[:, n_ctx:], g_q, be_q), 0.0)

    cat = jnp.concatenate(
        [d2.astype(_BF16), q.astype(_BF16), cg[:, :nc]], axis=-1)
    cmid = jnp.dot(cat, w_cat_ref[...], preferred_element_type=_F32)
    cmid = jnp.maximum(_gn1(cmid, g_c1, be_c1), 0.0)
    feat = jnp.dot(cmid.astype(_BF16), w_c2_ref[...],
                   preferred_element_type=_F32)

    ws = pl.multiple_of(ws_ref[b], 8)
    flag = flag_ref[b]
    hiv = hiv_ref[0]

    @pl.when(flag == 0)
    def _onehot_scatter():
        iota = lax.broadcasted_iota(jnp.int32, (_WIN, _TILE), 0)
        oh = (iota + ws == hiv).astype(_BF16)
        partial = jnp.dot(oh, feat.astype(_BF16), preferred_element_type=_F32)
        cur = acc_ref[0, pl.ds(ws, _WIN), :]
        acc_ref[0, pl.ds(ws, _WIN), :] = cur + partial

    @pl.when(flag != 0)
    def _row_scatter():
        feat_ref[...] = feat

        def body(qi, _):
            chunk = feat_ref[pl.ds(qi * 8, 8), :]
            for r in range(8):
                idx = his_ref[0, 0, qi * 8 + r]
                base = pl.multiple_of((idx >> 3) << 3, 8)
                sub = idx & 7
                mask = (lax.broadcasted_iota(jnp.int32, (8, 1), 0)
                        == sub).astype(_F32)
                cur = acc_ref[0, pl.ds(base, 8), :]
                acc_ref[0, pl.ds(base, 8), :] = cur + mask * chunk[r:r + 1, :]
            return 0

        lax.fori_loop(0, _TILE // 8, body, 0)


def _agt_kernel(acc_ref, agts_ref, wagt_ref, wl_ref, vec_ref, out_ref):
    v = vec_ref[...]
    g_n, be_n, g_l, be_l = v[0:1, :], v[1:2, :], v[2:3, :], v[3:4, :]

    res = agts_ref[...]
    added = acc_ref[0] + acc_ref[1]
    x = jnp.dot(res.astype(_BF16), wagt_ref[...],
                preferred_element_type=_F32) + added
    x = jnp.maximum(_gn1(x, g_n, be_n), 0.0)
    x = jnp.dot(x.astype(_BF16), wl_ref[...], preferred_element_type=_F32)
    x = _gn1(x, g_l, be_l)
    out_ref[...] = jnp.maximum(x + res, 0.0)


def _full_spec(shape):
    return pl.BlockSpec(shape, lambda c, j, ws, fl: (0,) * len(shape))


@jax.jit
def _att_forward(agts, agt_ctrs_cat, ctx, ctx_ctrs_cat, hi, wi, p):
    E = hi.shape[0]
    N, n_agt = agts.shape
    n_ctx = p["w_d2"].shape[0]
    nb = E // _TILE
    nblk = nb // 2

    hi_s, wi_s = lax.sort((hi, wi), num_keys=1)
    ag = jnp.concatenate(
        [agts, agt_ctrs_cat], axis=1).astype(_BF16)[hi_s]
    cg = jnp.concatenate(
        [ctx, ctx_ctrs_cat], axis=1).astype(_BF16)[wi_s]

    starts = hi_s[::_TILE]
    ws = jnp.minimum((starts >> 3) << 3, N - _WIN)
    last = hi_s[_TILE - 1::_TILE]
    flag = (last - ws >= _WIN).astype(jnp.int32)

    hiv = hi_s.reshape(nb, 1, _TILE)
    his = hi_s.reshape(nb, 1, _TILE)

    vec_ctx = jnp.concatenate(
        [p["w_d1"].T, p["b_d1"], p["g_d2"], p["be_d2"], p["g_q"], p["be_q"]],
        axis=0)
    vec_agt = jnp.concatenate([p["g_c1"], p["be_c1"]], axis=0)
    zc = jnp.zeros((n_ctx, n_ctx), _BF16)
    w_dq = jnp.concatenate([
        jnp.concatenate([p["w_d2"].T.astype(_BF16), zc], axis=1),
        jnp.concatenate([zc, p["w_q"].T.astype(_BF16)], axis=1)], axis=0)
    w_cat = jnp.concatenate(
        [p["w_c1d"].T, p["w_c1q"].T, p["w_c1c"].T], axis=0).astype(_BF16)
    weights = [vec_ctx, w_dq, w_cat, p["w_c2"].T.astype(_BF16), vec_agt]

    def row(ncol):
        return pl.BlockSpec((_TILE, ncol), lambda c, j, ws, fl: (c * nblk + j, 0))

    grid_spec = pltpu.PrefetchScalarGridSpec(
        num_scalar_prefetch=2,
        grid=(2, nblk),
        in_specs=[
            row(n_agt + 2), row(n_ctx + 2),
            pl.BlockSpec((1, 1, _TILE), lambda c, j, ws, fl: (c * nblk + j, 0, 0)),
            pl.BlockSpec((1, 1, _TILE), lambda c, j, ws, fl: (c * nblk + j, 0, 0),
                         memory_space=pltpu.SMEM),
        ] + [_full_spec(w.shape) for w in weights],
        out_specs=pl.BlockSpec((1, N, n_agt), lambda c, j, ws, fl: (c, 0, 0)),
        scratch_shapes=[pltpu.VMEM((_TILE, n_agt), _F32)],
    )
    acc = pl.pallas_call(
        _edge_kernel,
        grid_spec=grid_spec,
        out_shape=jax.ShapeDtypeStruct((2, N, n_agt), _F32),
        compiler_params=pltpu.CompilerParams(
            dimension_semantics=("parallel", "arbitrary")),
    )(ws, flag, ag, cg, hiv, his, *weights)

    tile_n = 1024
    vec = jnp.concatenate([p["g_n"], p["be_n"], p["g_l"], p["be_l"]], axis=0)
    out = pl.pallas_call(
        _agt_kernel,
        out_shape=jax.ShapeDtypeStruct((N, n_agt), _F32),
        grid=(N // tile_n,),
        in_specs=[
            pl.BlockSpec((2, tile_n, n_agt), lambda i: (0, i, 0)),
            pl.BlockSpec((tile_n, n_agt), lambda i: (i, 0)),
            pl.BlockSpec((n_agt, n_agt), lambda i: (0, 0)),
            pl.BlockSpec((n_agt, n_agt), lambda i: (0, 0)),
            pl.BlockSpec((4, n_agt), lambda i: (0, 0)),
        ],
        out_specs=pl.BlockSpec((tile_n, n_agt), lambda i: (i, 0)),
        compiler_params=pltpu.CompilerParams(
            dimension_semantics=("parallel",)),
    )(acc, agts, p["w_agt"].T.astype(_BF16), p["w_l"].T.astype(_BF16), vec)
    return out


def kernel(agts, ctx, agt_ctrs_cat, ctx_ctrs_cat, hi, wi,
           w_d1, b_d1, w_d2, g_d2, be_d2, w_q, g_q, be_q,
           w_c1d, w_c1q, w_c1c, g_c1, be_c1, w_c2, w_agt,
           g_n, be_n, w_l, g_l, be_l):
    p = {
        "w_d1": w_d1, "b_d1": b_d1, "w_d2": w_d2, "g_d2": g_d2, "be_d2": be_d2,
        "w_q": w_q, "g_q": g_q, "be_q": be_q,
        "w_c1d": w_c1d, "w_c1q": w_c1q, "w_c1c": w_c1c,
        "g_c1": g_c1, "be_c1": be_c1, "w_c2": w_c2,
        "w_agt": w_agt, "g_n": g_n, "be_n": be_n,
        "w_l": w_l, "g_l": g_l, "be_l": be_l,
    }
    return _att_forward(agts, agt_ctrs_cat, ctx, ctx_ctrs_cat, hi, wi, p)


# speedup vs baseline: 1.7672x; 1.0036x over previous
import jax
import jax.numpy as jnp
from jax import lax
from jax.experimental import pallas as pl
from jax.experimental.pallas import tpu as pltpu

_EPS = 1e-5
_BF16 = jnp.bfloat16
_F32 = jnp.float32

_TILE = 1024
_WIN = 512


def _gn1(x, gamma, beta):
    mu = jnp.mean(x, axis=-1, keepdims=True)
    var = jnp.mean((x - mu) ** 2, axis=-1, keepdims=True)
    return (x - mu) * lax.rsqrt(var + _EPS) * gamma + beta


def _edge_kernel(ws_ref, flag_ref, ag_ref, cg_ref,
                 hiv_ref, his_ref,
                 vec_ctx_ref, w_dq_ref, w_cat_ref, w_c2_ref, vec_agt_ref,
                 acc_ref, feat_ref):
    n_ctx = vec_ctx_ref.shape[1]
    nblk = pl.num_programs(1)
    c = pl.program_id(0)
    j = pl.program_id(1)
    b = c * nblk + j

    @pl.when(j == 0)
    def _init():
        acc_ref[...] = jnp.zeros_like(acc_ref)

    vc = vec_ctx_ref[...]
    wd1_0, wd1_1 = vc[0:1, :], vc[1:2, :]
    b_d1, g_d2, be_d2 = vc[2:3, :], vc[3:4, :], vc[4:5, :]
    g_q, be_q = vc[5:6, :], vc[6:7, :]
    va = vec_agt_ref[...]
    g_c1, be_c1 = va[0:1, :], va[1:2, :]

    na = ag_ref.shape[1] // 2
    nc = cg_ref.shape[1] // 2
    ag = ag_ref[...]
    cg = cg_ref[...]
    d = (ag[:, na:].astype(_F32) - cg[:, nc:].astype(_F32)) + b_d1
    d = jnp.maximum(d, 0.0)

    lhs = jnp.concatenate([d.astype(_BF16), ag[:, :na]], axis=-1)
    dq = jnp.dot(lhs, w_dq_ref[...], preferred_element_type=_F32)
    d2 = jnp.maximum(_gn1(dq[:, :n_ctx], g_d2, be_d2), 0.0)
    q = jnp.maximum(_gn1(dq[:, n_ctx:], g_q, be_q), 0.0)

    cat = jnp.concatenate(
        [d2.astype(_BF16), q.astype(_BF16), cg[:, :nc]], axis=-1)
    cmid = jnp.dot(cat, w_cat_ref[...], preferred_element_type=_F32)
    cmid = jnp.maximum(_gn1(cmid, g_c1, be_c1), 0.0)
    feat = jnp.dot(cmid.astype(_BF16), w_c2_ref[...],
                   preferred_element_type=_F32)

    ws = pl.multiple_of(ws_ref[b], 8)
    flag = flag_ref[b]
    hiv = hiv_ref[0]

    @pl.when(flag == 0)
    def _onehot_scatter():
        iota = lax.broadcasted_iota(jnp.int32, (_WIN, _TILE), 0)
        oh = (iota + ws == hiv).astype(_BF16)
        partial = jnp.dot(oh, feat.astype(_BF16), preferred_element_type=_F32)
        cur = acc_ref[0, pl.ds(ws, _WIN), :]
        acc_ref[0, pl.ds(ws, _WIN), :] = cur + partial

    @pl.when(flag != 0)
    def _row_scatter():
        feat_ref[...] = feat

        def body(qi, _):
            chunk = feat_ref[pl.ds(qi * 8, 8), :]
            for r in range(8):
                idx = his_ref[0, 0, qi * 8 + r]
                base = pl.multiple_of((idx >> 3) << 3, 8)
                sub = idx & 7
                mask = (lax.broadcasted_iota(jnp.int32, (8, 1), 0)
                        == sub).astype(_F32)
                cur = acc_ref[0, pl.ds(base, 8), :]
                acc_ref[0, pl.ds(base, 8), :] = cur + mask * chunk[r:r + 1, :]
            return 0

        lax.fori_loop(0, _TILE // 8, body, 0)


def _agt_kernel(acc_ref, agts_ref, wagt_ref, wl_ref, vec_ref, out_ref):
    v = vec_ref[...]
    g_n, be_n, g_l, be_l = v[0:1, :], v[1:2, :], v[2:3, :], v[3:4, :]

    res = agts_ref[...]
    added = acc_ref[0] + acc_ref[1]
    x = jnp.dot(res.astype(_BF16), wagt_ref[...],
                preferred_element_type=_F32) + added
    x = jnp.maximum(_gn1(x, g_n, be_n), 0.0)
    x = jnp.dot(x.astype(_BF16), wl_ref[...], preferred_element_type=_F32)
    x = _gn1(x, g_l, be_l)
    out_ref[...] = jnp.maximum(x + res, 0.0)


def _full_spec(shape):
    return pl.BlockSpec(shape, lambda c, j, ws, fl: (0,) * len(shape))


@jax.jit
def _att_forward(agts, agt_ctrs_cat, ctx, ctx_ctrs_cat, hi, wi, p):
    E = hi.shape[0]
    N, n_agt = agts.shape
    n_ctx = p["w_d2"].shape[0]
    nb = E // _TILE
    nblk = nb // 2

    hi_s, wi_s = lax.sort((hi, wi), num_keys=1)
    ag = jnp.concatenate(
        [agts, agt_ctrs_cat @ p["w_d1"].T], axis=1).astype(_BF16)[hi_s]
    cg = jnp.concatenate(
        [ctx, ctx_ctrs_cat @ p["w_d1"].T], axis=1).astype(_BF16)[wi_s]

    starts = hi_s[::_TILE]
    ws = jnp.minimum((starts >> 3) << 3, N - _WIN)
    last = hi_s[_TILE - 1::_TILE]
    flag = (last - ws >= _WIN).astype(jnp.int32)

    hiv = hi_s.reshape(nb, 1, _TILE)
    his = hi_s.reshape(nb, 1, _TILE)

    vec_ctx = jnp.concatenate(
        [p["w_d1"].T, p["b_d1"], p["g_d2"], p["be_d2"], p["g_q"], p["be_q"]],
        axis=0)
    vec_agt = jnp.concatenate([p["g_c1"], p["be_c1"]], axis=0)
    zc = jnp.zeros((n_ctx, n_ctx), _BF16)
    w_dq = jnp.concatenate([
        jnp.concatenate([p["w_d2"].T.astype(_BF16), zc], axis=1),
        jnp.concatenate([zc, p["w_q"].T.astype(_BF16)], axis=1)], axis=0)
    w_cat = jnp.concatenate(
        [p["w_c1d"].T, p["w_c1q"].T, p["w_c1c"].T], axis=0).astype(_BF16)
    weights = [vec_ctx, w_dq, w_cat, p["w_c2"].T.astype(_BF16), vec_agt]

    def row(ncol):
        return pl.BlockSpec((_TILE, ncol), lambda c, j, ws, fl: (c * nblk + j, 0))

    grid_spec = pltpu.PrefetchScalarGridSpec(
        num_scalar_prefetch=2,
        grid=(2, nblk),
        in_specs=[
            row(2 * n_agt), row(2 * n_ctx),
            pl.BlockSpec((1, 1, _TILE), lambda c, j, ws, fl: (c * nblk + j, 0, 0)),
            pl.BlockSpec((1, 1, _TILE), lambda c, j, ws, fl: (c * nblk + j, 0, 0),
                         memory_space=pltpu.SMEM),
        ] + [_full_spec(w.shape) for w in weights],
        out_specs=pl.BlockSpec((1, N, n_agt), lambda c, j, ws, fl: (c, 0, 0)),
        scratch_shapes=[pltpu.VMEM((_TILE, n_agt), _F32)],
    )
    acc = pl.pallas_call(
        _edge_kernel,
        grid_spec=grid_spec,
        out_shape=jax.ShapeDtypeStruct((2, N, n_agt), _F32),
        compiler_params=pltpu.CompilerParams(
            dimension_semantics=("parallel", "arbitrary")),
    )(ws, flag, ag, cg, hiv, his, *weights)

    tile_n = 1024
    vec = jnp.concatenate([p["g_n"], p["be_n"], p["g_l"], p["be_l"]], axis=0)
    out = pl.pallas_call(
        _agt_kernel,
        out_shape=jax.ShapeDtypeStruct((N, n_agt), _F32),
        grid=(N // tile_n,),
        in_specs=[
            pl.BlockSpec((2, tile_n, n_agt), lambda i: (0, i, 0)),
            pl.BlockSpec((tile_n, n_agt), lambda i: (i, 0)),
            pl.BlockSpec((n_agt, n_agt), lambda i: (0, 0)),
            pl.BlockSpec((n_agt, n_agt), lambda i: (0, 0)),
            pl.BlockSpec((4, n_agt), lambda i: (0, 0)),
        ],
        out_specs=pl.BlockSpec((tile_n, n_agt), lambda i: (i, 0)),
        compiler_params=pltpu.CompilerParams(
            dimension_semantics=("parallel",)),
    )(acc, agts, p["w_agt"].T.astype(_BF16), p["w_l"].T.astype(_BF16), vec)
    return out


def kernel(agts, ctx, agt_ctrs_cat, ctx_ctrs_cat, hi, wi,
           w_d1, b_d1, w_d2, g_d2, be_d2, w_q, g_q, be_q,
           w_c1d, w_c1q, w_c1c, g_c1, be_c1, w_c2, w_agt,
           g_n, be_n, w_l, g_l, be_l):
    p = {
        "w_d1": w_d1, "b_d1": b_d1, "w_d2": w_d2, "g_d2": g_d2, "be_d2": be_d2,
        "w_q": w_q, "g_q": g_q, "be_q": be_q,
        "w_c1d": w_c1d, "w_c1q": w_c1q, "w_c1c": w_c1c,
        "g_c1": g_c1, "be_c1": be_c1, "w_c2": w_c2,
        "w_agt": w_agt, "g_n": g_n, "be_n": be_n,
        "w_l": w_l, "g_l": g_l, "be_l": be_l,
    }
    return _att_forward(agts, agt_ctrs_cat, ctx, ctx_ctrs_cat, hi, wi, p)
